# baseline (speedup 1.0000x reference)
import functools

import jax
import jax.numpy as jnp
from jax import lax
from jax.experimental import pallas as pl
from jax.experimental.pallas import tpu as pltpu

_EMB = 24
_HID = 96
_NREL = 3
_NSHAPE = 8
_NCOLOR = 8
_MAXPOS = 25
_NCLASS = 16

_NB = 2000
_EB = 4000
_CCH = 32


def _onehot(idx, k):
    cols = lax.broadcasted_iota(jnp.int32, (idx.shape[0], k), 1)
    return (idx[:, None] == cols).astype(jnp.float32)


def _embed_kernel(x_ref, es_ref, ec_ref, ep_ref, out_ref):
    xs = x_ref[:, 0]
    xc = x_ref[:, 1]
    xp = jnp.minimum(x_ref[:, 2], _MAXPOS - 1)
    out_ref[:, 0:_EMB] = jnp.dot(_onehot(xs, _NSHAPE), es_ref[...],
                                 preferred_element_type=jnp.float32)
    out_ref[:, _EMB:2 * _EMB] = jnp.dot(_onehot(xc, _NCOLOR), ec_ref[...],
                                        preferred_element_type=jnp.float32)
    out_ref[:, 2 * _EMB:3 * _EMB] = jnp.dot(_onehot(xp, _MAXPOS), ep_ref[...],
                                            preferred_element_type=jnp.float32)
    ones_col = jnp.ones((out_ref.shape[0], 1), jnp.float32)
    out_ref[:, 3 * _EMB:3 * _EMB + 1] = ones_col
    out_ref[:, 3 * _EMB + 1:] = jnp.zeros(
        (out_ref.shape[0], out_ref.shape[1] - 3 * _EMB - 1), jnp.float32)


def _scatter_kernel(src_ref, dst_ref, hw_ref, acc_ref):
    @pl.when(pl.program_id(0) == 0)
    def _():
        acc_ref[...] = jnp.zeros_like(acc_ref)

    def body(i, carry):
        s = src_ref[0, 0, i]
        d = dst_ref[0, 0, i]
        acc_ref[pl.ds(d, 1), :] = acc_ref[pl.ds(d, 1), :] + hw_ref[pl.ds(s, 1), :]
        return carry

    lax.fori_loop(0, _EB, body, 0, unroll=8)


def _dense_kernel(h_ref, a0_ref, a1_ref, a2_ref, inv_ref, wroot_ref, wrel_ref,
                  b_ref, out_ref, *, d_in):
    h = h_ref[:, 0:d_in]
    out = jnp.dot(h, wroot_ref[...], preferred_element_type=jnp.float32)
    out = out + b_ref[0:1, :]
    for r, a_ref in enumerate((a0_ref, a1_ref, a2_ref)):
        mean_r = a_ref[:, 0:d_in] * inv_ref[:, r][:, None]
        out = out + jnp.dot(mean_r, wrel_ref[r], preferred_element_type=jnp.float32)
    out_ref[...] = jnp.maximum(out, 0.0)


def _pool_kernel(h_ref, batch_ref, linw_ref, linb_ref,
                 sum_ref, cnt_ref, logit_ref, *, num_graphs, n_blocks):
    i = pl.program_id(0)

    @pl.when(i == 0)
    def _():
        sum_ref[...] = jnp.zeros_like(sum_ref)
        cnt_ref[...] = jnp.zeros_like(cnt_ref)

    m = _onehot(batch_ref[:, 0], num_graphs)
    sum_ref[...] = sum_ref[...] + lax.dot_general(
        m, h_ref[...], (((0,), (0,)), ((), ())),
        preferred_element_type=jnp.float32)
    ones = jnp.ones((h_ref.shape[0], cnt_ref.shape[1]), jnp.float32)
    cnt_ref[...] = cnt_ref[...] + lax.dot_general(
        m, ones, (((0,), (0,)), ((), ())),
        preferred_element_type=jnp.float32)

    @pl.when(i == n_blocks - 1)
    def _():
        cnt = jnp.maximum(cnt_ref[:, 0:1], 1.0)
        hg = sum_ref[...] / cnt
        logit_ref[...] = jnp.dot(hg, linw_ref[...],
                                 preferred_element_type=jnp.float32) + linb_ref[0:1, :]


def _aggregate(h_pad, src3, dst3_per_rel, n):
    n_eb = src3.shape[0]
    outs = []
    for dst3 in dst3_per_rel:
        acc = pl.pallas_call(
            _scatter_kernel,
            grid=(n_eb,),
            in_specs=[
                pl.BlockSpec((1, 1, _EB), lambda e: (e, 0, 0),
                             memory_space=pltpu.SMEM),
                pl.BlockSpec((1, 1, _EB), lambda e: (e, 0, 0),
                             memory_space=pltpu.SMEM),
                pl.BlockSpec((n, _HID), lambda e: (0, 0)),
            ],
            out_specs=pl.BlockSpec((n + 8, _HID), lambda e: (0, 0)),
            out_shape=jax.ShapeDtypeStruct((n + 8, _HID), jnp.float32),
        )(src3, dst3, h_pad)
        outs.append(acc[:n])
    return outs


def _rgcn_layer(h_pad, aggs, inv_t, w_root, w_rel, b, d_in, n):
    n_nb = n // _NB
    dense = functools.partial(_dense_kernel, d_in=d_in)
    out = pl.pallas_call(
        dense,
        grid=(n_nb,),
        in_specs=[
            pl.BlockSpec((_NB, _HID), lambda i: (i, 0)),
            pl.BlockSpec((_NB, _HID), lambda i: (i, 0)),
            pl.BlockSpec((_NB, _HID), lambda i: (i, 0)),
            pl.BlockSpec((_NB, _HID), lambda i: (i, 0)),
            pl.BlockSpec((_NB, _NREL), lambda i: (i, 0)),
            pl.BlockSpec((d_in, _HID), lambda i: (0, 0)),
            pl.BlockSpec((_NREL, d_in, _HID), lambda i: (0, 0, 0)),
            pl.BlockSpec((1, _HID), lambda i: (0, 0)),
        ],
        out_specs=pl.BlockSpec((_NB, _HID), lambda i: (i, 0)),
        out_shape=jax.ShapeDtypeStruct((n, _HID), jnp.float32),
    )(h_pad, aggs[0], aggs[1], aggs[2], inv_t, w_root, w_rel,
      b.reshape(1, _HID))
    return out


def kernel(x, edge_index, edge_type, batch, emb_shape, emb_color, emb_pos,
           W_root1, W_rel1, b1, W_root2, W_rel2, b2, lin_W, lin_b):
    n = x.shape[0]
    e = edge_index.shape[1]
    num_graphs = 512
    n_nb = n // _NB
    n_eb = e // _EB

    src3 = edge_index[0].reshape(n_eb, 1, _EB)
    dst3_per_rel = [
        jnp.where(edge_type == r, edge_index[1], n).reshape(n_eb, 1, _EB)
        for r in range(_NREL)
    ]

    h_pad = pl.pallas_call(
        _embed_kernel,
        grid=(n_nb,),
        in_specs=[
            pl.BlockSpec((_NB, 3), lambda i: (i, 0)),
            pl.BlockSpec((_NSHAPE, _EMB), lambda i: (0, 0)),
            pl.BlockSpec((_NCOLOR, _EMB), lambda i: (0, 0)),
            pl.BlockSpec((_MAXPOS, _EMB), lambda i: (0, 0)),
        ],
        out_specs=pl.BlockSpec((_NB, _HID), lambda i: (i, 0)),
        out_shape=jax.ShapeDtypeStruct((n, _HID), jnp.float32),
    )(x, emb_shape, emb_color, emb_pos)

    aggs1 = _aggregate(h_pad, src3, dst3_per_rel, n)
    inv_t = jnp.stack(
        [1.0 / jnp.maximum(a[:, 3 * _EMB], 1.0) for a in aggs1], axis=1)

    h1 = _rgcn_layer(h_pad, aggs1, inv_t, W_root1, W_rel1, b1, 3 * _EMB, n)

    aggs2 = _aggregate(h1, src3, dst3_per_rel, n)
    h2 = _rgcn_layer(h1, aggs2, inv_t, W_root2, W_rel2, b2, _HID, n)

    pool = functools.partial(_pool_kernel, num_graphs=num_graphs, n_blocks=n_nb)
    _, _, logits = pl.pallas_call(
        pool,
        grid=(n_nb,),
        in_specs=[
            pl.BlockSpec((_NB, _HID), lambda i: (i, 0)),
            pl.BlockSpec((_NB, 1), lambda i: (i, 0)),
            pl.BlockSpec((_HID, _NCLASS), lambda i: (0, 0)),
            pl.BlockSpec((1, _NCLASS), lambda i: (0, 0)),
        ],
        out_specs=[
            pl.BlockSpec((num_graphs, _HID), lambda i: (0, 0)),
            pl.BlockSpec((num_graphs, 128), lambda i: (0, 0)),
            pl.BlockSpec((num_graphs, _NCLASS), lambda i: (0, 0)),
        ],
        out_shape=[
            jax.ShapeDtypeStruct((num_graphs, _HID), jnp.float32),
            jax.ShapeDtypeStruct((num_graphs, 128), jnp.float32),
            jax.ShapeDtypeStruct((num_graphs, _NCLASS), jnp.float32),
        ],
    )(h2, batch.reshape(n, 1), lin_W, lin_b.reshape(1, _NCLASS))
    return logits
